# baseline (speedup 1.0000x reference)
import functools

import jax
import jax.numpy as jnp
from jax import lax
from jax.experimental import pallas as pl
from jax.experimental.pallas import tpu as pltpu
from jax.experimental.pallas import tpu_sc as plsc

S, M, A, H, DIN = 32, 256, 64, 64, 128
N = 262144
NSLOT = S * M * A
PAD = 1024
BLK = 2048

_info = plsc.get_sparse_core_info()
NC, NS, L = _info.num_cores, _info.num_subcores, _info.num_lanes
NW = NC * NS
R = NSLOT // NW
CHUNK = 4096
GB = 128


def _mm_body(x_ref, w_ref, o_ref):
    i = pl.program_id(0)
    nb = pl.num_programs(0)

    @pl.when(i < nb - 1)
    def _compute():
        y = jnp.tanh(
            jnp.dot(x_ref[...], w_ref[...], preferred_element_type=jnp.float32)
        )
        o_ref[...] = jnp.concatenate([y, jnp.zeros_like(y)], axis=1)

    @pl.when(i == nb - 1)
    def _zero_pad():
        o_ref[...] = jnp.zeros_like(o_ref)


def _encode_vals(feats, w):
    nx = N // BLK
    return pl.pallas_call(
        _mm_body,
        grid=(nx + 1,),
        in_specs=[
            pl.BlockSpec((BLK, DIN), lambda i: (jnp.minimum(i, nx - 1), 0)),
            pl.BlockSpec((DIN, H), lambda i: (0, 0)),
        ],
        out_specs=pl.BlockSpec((BLK, 2 * H), lambda i: (i, 0)),
        out_shape=jax.ShapeDtypeStruct((N + PAD, 2 * H), jnp.float32),
    )(feats, w)


UNROLL = 8
NCHUNK = N // CHUNK
GN = R // GB
RING = 4


def _winner_body(sidx_hbm, nidx_hbm, aidx_hbm, winner_hbm,
                 sbufs, nbufs, abufs, insems, winner):
    wid = lax.axis_index("s") * NC + lax.axis_index("c")
    base = wid * R

    def _init(t, _):
        v = jnp.full((L,), 2 * N + 2 * ((t * L) & (PAD - 1)), jnp.int32) + (
            lax.iota(jnp.int32, L) * 2
        )
        winner[pl.ds(t * L, L)] = v
        return 0

    lax.fori_loop(0, R // L, _init, 0)

    def _stage(c, p):
        cb = c * CHUNK
        pltpu.async_copy(sidx_hbm.at[pl.ds(cb, CHUNK)], sbufs[p], insems[p])
        pltpu.async_copy(nidx_hbm.at[pl.ds(cb, CHUNK)], nbufs[p], insems[p])
        pltpu.async_copy(aidx_hbm.at[pl.ds(cb, CHUNK)], abufs[p], insems[p])

    def _wait_stage(p):
        pltpu.make_async_copy(sidx_hbm.at[pl.ds(0, CHUNK)], sbufs[p], insems[p]).wait()
        pltpu.make_async_copy(nidx_hbm.at[pl.ds(0, CHUNK)], nbufs[p], insems[p]).wait()
        pltpu.make_async_copy(aidx_hbm.at[pl.ds(0, CHUNK)], abufs[p], insems[p]).wait()

    def _scan_chunk(c, p):
        cbase = c * CHUNK

        def _group(t, _):
            for u in range(UNROLL):
                off = t * (L * UNROLL) + u * L
                sv = sbufs[p][pl.ds(off, L)]
                nv = nbufs[p][pl.ds(off, L)]
                av = abufs[p][pl.ds(off, L)]
                flat = sv * (M * A) + nv * A + av
                occ = jnp.full((L,), 2 * (cbase + off), jnp.int32) + (
                    lax.iota(jnp.int32, L) * 2
                )
                rel = flat - base
                mask = plsc.bitcast(rel, jnp.uint32) < jnp.uint32(R)
                loc = rel & (R - 1)
                plsc.store_scatter(winner, [loc], occ, mask=mask)
            return 0

        lax.fori_loop(0, CHUNK // (L * UNROLL), _group, 0)

    _stage(0, 0)
    _stage(1, 1)

    def _chunk_pair(cp, _):
        for p in range(2):
            c = cp * 2 + p
            _wait_stage(p)
            _scan_chunk(c, p)

            @pl.when(c + 2 < NCHUNK)
            def _():
                _stage(c + 2, p)
        return 0

    lax.fori_loop(0, NCHUNK // 2, _chunk_pair, 0)

    pltpu.sync_copy(winner, winner_hbm.at[pl.ds(base, R)])


def _gather_body(winner_hbm, vals_hbm, out_hbm, winner, rows, gsems, wsems):
    wid = lax.axis_index("s") * NC + lax.axis_index("c")
    base = wid * R
    pltpu.sync_copy(winner_hbm.at[pl.ds(base, R)], winner)

    LAG = 2

    def _start_gather(g, b):
        pltpu.async_copy(vals_hbm.at[winner.at[pl.ds(g * GB, GB)]], rows[b], gsems[b])

    def _wait_gather(b):
        pltpu.make_async_copy(vals_hbm.at[winner.at[pl.ds(0, GB)]], rows[b], gsems[b]).wait()

    def _start_write(g, b):
        pltpu.async_copy(rows[b], out_hbm.at[pl.ds(base + g * GB, GB)], wsems[b])

    def _wait_write(b):
        pltpu.make_async_copy(rows[b], out_hbm.at[pl.ds(0, GB)], wsems[b]).wait()

    def _ring_iter(k, first):
        for b in range(RING):
            g = k * RING + b
            if not first:
                _wait_write(b)
            _start_gather(g, b)
            gr = g - LAG
            br = (b - LAG) % RING
            if not (first and b < LAG):
                _wait_gather(br)
                _start_write(gr, br)
        return 0

    _ring_iter(0, True)
    lax.fori_loop(1, GN // RING, lambda k, _: _ring_iter(k, False), 0)
    for i in range(LAG):
        g = GN - LAG + i
        b = g % RING
        _wait_gather(b)
        _start_write(g, b)
    for b in range(RING):
        _wait_write(b)


_sc_mesh = plsc.VectorSubcoreMesh(core_axis_name="c", subcore_axis_name="s")
_sc_params = pltpu.CompilerParams(
    needs_layout_passes=False, use_tc_tiling_on_sc=False
)

_winner_kernel = pl.kernel(
    _winner_body,
    out_type=jax.ShapeDtypeStruct((NSLOT,), jnp.int32),
    mesh=_sc_mesh,
    compiler_params=_sc_params,
    scratch_types=[
        [pltpu.VMEM((CHUNK,), jnp.int32) for _ in range(2)],
        [pltpu.VMEM((CHUNK,), jnp.int32) for _ in range(2)],
        [pltpu.VMEM((CHUNK,), jnp.int32) for _ in range(2)],
        [pltpu.SemaphoreType.DMA for _ in range(2)],
        pltpu.VMEM((R,), jnp.int32),
    ],
)

_gather_kernel = pl.kernel(
    _gather_body,
    out_type=jax.ShapeDtypeStruct((NSLOT, H), jnp.float32),
    mesh=_sc_mesh,
    compiler_params=_sc_params,
    scratch_types=[
        pltpu.VMEM((R,), jnp.int32),
        [pltpu.VMEM((GB, H), jnp.float32) for _ in range(RING)],
        [pltpu.SemaphoreType.DMA for _ in range(RING)],
        [pltpu.SemaphoreType.DMA for _ in range(RING)],
    ],
)


def _tr_body(x_ref, o_ref):
    ident = jnp.eye(2 * H, dtype=jnp.float32)
    xt = lax.dot_general(
        x_ref[0], ident, (((0,), (0,)), ((), ())),
        precision=lax.Precision.HIGHEST,
        preferred_element_type=jnp.float32,
    )
    o_ref[0] = xt.reshape(A * H, 2 * H)


def _to_final(gout):
    g4 = gout.reshape(S * M // 128, 128, 32, 128)
    out3 = pl.pallas_call(
        _tr_body,
        grid=(S, M // 128),
        in_specs=[
            pl.BlockSpec((1, 128, 32, 128), lambda s, mb: (s * 2 + mb, 0, 0, 0)),
        ],
        out_specs=pl.BlockSpec((1, A * H, 128), lambda s, mb: (s, 0, mb)),
        out_shape=jax.ShapeDtypeStruct((S, A * H, M), jnp.float32),
    )(g4)
    return out3.reshape(S, A, H, M).transpose(0, 3, 1, 2)


@jax.jit
def kernel(attrib_value_feats, W, sample_idx, node_idx, attrib_idx):
    winner = _winner_kernel(sample_idx, node_idx, attrib_idx)
    vals_wide = _encode_vals(attrib_value_feats, W)
    table = vals_wide.reshape(2 * (N + PAD), H)
    out = _gather_kernel(winner, table)
    return _to_final(out)


# speedup vs baseline: 1.1066x; 1.1066x over previous
import functools

import jax
import jax.numpy as jnp
from jax import lax
from jax.experimental import pallas as pl
from jax.experimental.pallas import tpu as pltpu
from jax.experimental.pallas import tpu_sc as plsc

S, M, A, H, DIN = 32, 256, 64, 64, 128
N = 262144
NSLOT = S * M * A
PAD = 1024
BLK = 4096

_info = plsc.get_sparse_core_info()
NC, NS, L = _info.num_cores, _info.num_subcores, _info.num_lanes
NW = NC * NS
R = NSLOT // NW
CHUNK = 8192
GB = 128


def _mm_body(x_ref, w_ref, o_ref):
    i = pl.program_id(0)
    nb = pl.num_programs(0)

    @pl.when(i < nb - 1)
    def _compute():
        y = jnp.tanh(
            jnp.dot(x_ref[...], w_ref[...], preferred_element_type=jnp.float32)
        )
        o_ref[...] = jnp.concatenate([y, jnp.zeros_like(y)], axis=1)

    @pl.when(i == nb - 1)
    def _zero_pad():
        o_ref[...] = jnp.zeros_like(o_ref)


def _encode_vals(feats, w):
    nx = N // BLK
    return pl.pallas_call(
        _mm_body,
        grid=(nx + 1,),
        in_specs=[
            pl.BlockSpec((BLK, DIN), lambda i: (jnp.minimum(i, nx - 1), 0)),
            pl.BlockSpec((DIN, H), lambda i: (0, 0)),
        ],
        out_specs=pl.BlockSpec((BLK, 2 * H), lambda i: (i, 0)),
        out_shape=jax.ShapeDtypeStruct((N + PAD, 2 * H), jnp.float32),
    )(feats, w)


UNROLL = 16
NCHUNK = N // CHUNK
GN = R // GB
RING = 4


def _winner_body(sidx_hbm, nidx_hbm, aidx_hbm, winner_hbm,
                 sbufs, nbufs, abufs, insems, winner):
    wid = lax.axis_index("s") * NC + lax.axis_index("c")
    base = wid * R

    def _init(t, _):
        v = jnp.full((L,), 2 * N + 2 * ((t * L) & (PAD - 1)), jnp.int32) + (
            lax.iota(jnp.int32, L) * 2
        )
        winner[pl.ds(t * L, L)] = v
        return 0

    lax.fori_loop(0, R // L, _init, 0)

    def _stage(c, p):
        cb = c * CHUNK
        pltpu.async_copy(sidx_hbm.at[pl.ds(cb, CHUNK)], sbufs[p], insems[p])
        pltpu.async_copy(nidx_hbm.at[pl.ds(cb, CHUNK)], nbufs[p], insems[p])
        pltpu.async_copy(aidx_hbm.at[pl.ds(cb, CHUNK)], abufs[p], insems[p])

    def _wait_stage(p):
        pltpu.make_async_copy(sidx_hbm.at[pl.ds(0, CHUNK)], sbufs[p], insems[p]).wait()
        pltpu.make_async_copy(nidx_hbm.at[pl.ds(0, CHUNK)], nbufs[p], insems[p]).wait()
        pltpu.make_async_copy(aidx_hbm.at[pl.ds(0, CHUNK)], abufs[p], insems[p]).wait()

    def _scan_chunk(c, p):
        cbase = c * CHUNK

        def _group(t, _):
            for u in range(UNROLL):
                off = t * (L * UNROLL) + u * L
                sv = sbufs[p][pl.ds(off, L)]
                nv = nbufs[p][pl.ds(off, L)]
                av = abufs[p][pl.ds(off, L)]
                flat = sv * (M * A) + nv * A + av
                occ = jnp.full((L,), 2 * (cbase + off), jnp.int32) + (
                    lax.iota(jnp.int32, L) * 2
                )
                rel = flat - base
                mask = plsc.bitcast(rel, jnp.uint32) < jnp.uint32(R)
                loc = rel & (R - 1)
                plsc.store_scatter(winner, [loc], occ, mask=mask)
            return 0

        lax.fori_loop(0, CHUNK // (L * UNROLL), _group, 0)

    _stage(0, 0)
    _stage(1, 1)

    def _chunk_pair(cp, _):
        for p in range(2):
            c = cp * 2 + p
            _wait_stage(p)
            _scan_chunk(c, p)

            @pl.when(c + 2 < NCHUNK)
            def _():
                _stage(c + 2, p)
        return 0

    lax.fori_loop(0, NCHUNK // 2, _chunk_pair, 0)

    pltpu.sync_copy(winner, winner_hbm.at[pl.ds(base, R)])


def _gather_body(winner_hbm, vals_hbm, out_hbm, winner, rows, gsems, wsems):
    wid = lax.axis_index("s") * NC + lax.axis_index("c")
    base = wid * R
    pltpu.sync_copy(winner_hbm.at[pl.ds(base, R)], winner)

    LAG = 2

    def _start_gather(g, b):
        pltpu.async_copy(vals_hbm.at[winner.at[pl.ds(g * GB, GB)]], rows[b], gsems[b])

    def _wait_gather(b):
        pltpu.make_async_copy(vals_hbm.at[winner.at[pl.ds(0, GB)]], rows[b], gsems[b]).wait()

    def _start_write(g, b):
        pltpu.async_copy(rows[b], out_hbm.at[pl.ds(base + g * GB, GB)], wsems[b])

    def _wait_write(b):
        pltpu.make_async_copy(rows[b], out_hbm.at[pl.ds(0, GB)], wsems[b]).wait()

    def _ring_iter(k, first):
        for b in range(RING):
            g = k * RING + b
            if not first:
                _wait_write(b)
            _start_gather(g, b)
            gr = g - LAG
            br = (b - LAG) % RING
            if not (first and b < LAG):
                _wait_gather(br)
                _start_write(gr, br)
        return 0

    _ring_iter(0, True)
    lax.fori_loop(1, GN // RING, lambda k, _: _ring_iter(k, False), 0)
    for i in range(LAG):
        g = GN - LAG + i
        b = g % RING
        _wait_gather(b)
        _start_write(g, b)
    for b in range(RING):
        _wait_write(b)


_sc_mesh = plsc.VectorSubcoreMesh(core_axis_name="c", subcore_axis_name="s")
_sc_params = pltpu.CompilerParams(
    needs_layout_passes=False, use_tc_tiling_on_sc=False
)

_winner_kernel = pl.kernel(
    _winner_body,
    out_type=jax.ShapeDtypeStruct((NSLOT,), jnp.int32),
    mesh=_sc_mesh,
    compiler_params=_sc_params,
    scratch_types=[
        [pltpu.VMEM((CHUNK,), jnp.int32) for _ in range(2)],
        [pltpu.VMEM((CHUNK,), jnp.int32) for _ in range(2)],
        [pltpu.VMEM((CHUNK,), jnp.int32) for _ in range(2)],
        [pltpu.SemaphoreType.DMA for _ in range(2)],
        pltpu.VMEM((R,), jnp.int32),
    ],
)

_gather_kernel = pl.kernel(
    _gather_body,
    out_type=jax.ShapeDtypeStruct((NSLOT, H), jnp.float32),
    mesh=_sc_mesh,
    compiler_params=_sc_params,
    scratch_types=[
        pltpu.VMEM((R,), jnp.int32),
        [pltpu.VMEM((GB, H), jnp.float32) for _ in range(RING)],
        [pltpu.SemaphoreType.DMA for _ in range(RING)],
        [pltpu.SemaphoreType.DMA for _ in range(RING)],
    ],
)


def _tr_body(x_ref, o_ref):
    ident = jnp.eye(2 * H, dtype=jnp.float32)
    xt = lax.dot_general(
        x_ref[0], ident, (((0,), (0,)), ((), ())),
        preferred_element_type=jnp.float32,
    )
    o_ref[0] = xt.reshape(A * H, 2 * H)


def _to_final(gout):
    g4 = gout.reshape(S * M // 128, 128, 32, 128)
    out3 = pl.pallas_call(
        _tr_body,
        grid=(S, M // 128),
        in_specs=[
            pl.BlockSpec((1, 128, 32, 128), lambda s, mb: (s * 2 + mb, 0, 0, 0)),
        ],
        out_specs=pl.BlockSpec((1, A * H, 128), lambda s, mb: (s, 0, mb)),
        out_shape=jax.ShapeDtypeStruct((S, A * H, M), jnp.float32),
    )(g4)
    return out3.reshape(S, A, H, M).transpose(0, 3, 1, 2)


@jax.jit
def kernel(attrib_value_feats, W, sample_idx, node_idx, attrib_idx):
    winner = _winner_kernel(sample_idx, node_idx, attrib_idx)
    vals_wide = _encode_vals(attrib_value_feats, W)
    table = vals_wide.reshape(2 * (N + PAD), H)
    out = _gather_kernel(winner, table)
    return _to_final(out)
